# baseline (speedup 1.0000x reference)
import functools

import jax
import jax.numpy as jnp
from jax import lax
from jax.experimental import pallas as pl
from jax.experimental.pallas import tpu as pltpu
from jax.experimental.pallas import tpu_sc as plsc

S = 2048
D = 1024
E = 8
H = 4096
TOPK = 2
A = S * TOPK

BLK = 256
MAXB = A // BLK + E
PAD_N = MAXB * BLK
F = 1024
NF = H // F

NC = 2
NS = 16
NW = NC * NS

G_CH = 32
G_NCH = (A // NW) // G_CH
C_TOK = S // NW
C_CH = 32


def _router(x2d, Wr, br):
  def body(x_ref, wr_ref, br_ref, e_ref, w_ref):
    logits = jnp.dot(x_ref[...], wr_ref[...],
                     preferred_element_type=jnp.float32) + br_ref[...]
    ids = lax.broadcasted_iota(jnp.int32, (S, E), 1)
    neg = jnp.float32(-3.0e38)
    m0 = jnp.max(logits, axis=-1, keepdims=True)
    i0 = jnp.min(jnp.where(logits == m0, ids, E), axis=-1, keepdims=True)
    masked = jnp.where(ids == i0, neg, logits)
    m1 = jnp.max(masked, axis=-1, keepdims=True)
    i1 = jnp.min(jnp.where(masked == m1, ids, E), axis=-1, keepdims=True)
    w0 = 1.0 / (1.0 + jnp.exp(m1 - m0))
    e_ref[...] = jnp.concatenate([i0, i1], axis=1)
    w_ref[...] = jnp.concatenate([w0, 1.0 - w0], axis=1)

  out_shape = (
      jax.ShapeDtypeStruct((S, TOPK), jnp.int32),
      jax.ShapeDtypeStruct((S, TOPK), jnp.float32),
  )
  return pl.pallas_call(body, out_shape=out_shape)(x2d, Wr, br.reshape(1, E))


def _dispatch_tables(e2, w2):
  i32 = jnp.int32
  flat_e = e2.reshape(A)
  flat_w = w2.reshape(A)
  onehot = (flat_e[:, None] == jnp.arange(E, dtype=i32)[None, :]).astype(i32)
  ccum = jnp.cumsum(onehot, axis=0)
  g = ccum[-1]
  rank = jnp.take_along_axis(ccum, flat_e[:, None], axis=1)[:, 0] - 1
  nblk = (g + BLK - 1) // BLK
  bcum = jnp.cumsum(nblk).astype(i32)
  bcum_ex = jnp.concatenate([jnp.zeros(1, i32), bcum[:-1]])
  total_blocks = bcum[-1]

  b_ids = jnp.arange(MAXB, dtype=i32)
  eb = (b_ids[:, None] >= bcum[None, :]).sum(axis=1, dtype=i32)
  active = (b_ids < total_blocks).astype(i32)
  e_last = jnp.max(jnp.where(g > 0, jnp.arange(E, dtype=i32), 0))
  eb_safe = jnp.where(active == 1, jnp.clip(eb, 0, E - 1), e_last)

  dst_pad = (bcum_ex[flat_e] + rank // BLK) * BLK + rank % BLK

  pp = dst_pad.reshape(S, TOPK)
  tok_g = (jnp.arange(A, dtype=i32) // TOPK).reshape(NW, G_NCH, G_CH)
  dst_g = dst_pad.reshape(NW, G_NCH, G_CH)
  w8 = jnp.broadcast_to(flat_w[:, None], (A, 128))
  return tok_g, dst_g, w8, pp[:, 0], pp[:, 1], eb_safe, active


def _gather_dispatch(x2d, tok_g, dst_g, w8):
  mesh = plsc.VectorSubcoreMesh(core_axis_name="c", subcore_axis_name="s")

  @functools.partial(
      pl.kernel, mesh=mesh,
      out_type=(jax.ShapeDtypeStruct((PAD_N, D // 2), jnp.int32),
                jax.ShapeDtypeStruct((PAD_N, 128), jnp.float32)),
      scratch_types=[
          pltpu.VMEM((G_NCH, G_CH), jnp.int32),
          pltpu.VMEM((G_NCH, G_CH), jnp.int32),
          pltpu.VMEM((G_NCH * G_CH,), jnp.int32),
          pltpu.VMEM((G_NCH * G_CH, 128), jnp.float32),
          pltpu.VMEM((G_CH, D // 2), jnp.int32),
          pltpu.VMEM((G_CH, D // 2), jnp.int32),
          pltpu.VMEM((G_CH, D // 2), jnp.int32),
          pltpu.SemaphoreType.DMA,
          pltpu.SemaphoreType.DMA,
          pltpu.SemaphoreType.DMA,
          pltpu.SemaphoreType.DMA,
          pltpu.SemaphoreType.DMA,
          pltpu.SemaphoreType.DMA,
          pltpu.SemaphoreType.DMA,
      ],
  )
  def k(x_hbm, tok_hbm, dst_hbm, dstf_hbm, w8_hbm, xpad_hbm, wpad_hbm,
        tok_v, dst_v, wdst_v, w_v, r0, r1, r2, gs0, gs1, gs2, ss0, ss1, ss2,
        ws):
    wid = lax.axis_index("s") * NC + lax.axis_index("c")
    pltpu.sync_copy(tok_hbm.at[wid], tok_v)
    pltpu.sync_copy(dst_hbm.at[wid], dst_v)
    pltpu.sync_copy(dstf_hbm.at[wid], wdst_v)
    pltpu.sync_copy(w8_hbm.at[pl.ds(wid * (G_NCH * G_CH), G_NCH * G_CH)], w_v)
    rows = (r0, r1, r2)
    gsem = (gs0, gs1, gs2)
    ssem = (ss0, ss1, ss2)

    def gather(c, slot):
      return pltpu.async_copy(x_hbm.at[tok_v.at[c]], rows[slot], gsem[slot])

    def scatter(c, slot):
      return pltpu.async_copy(rows[slot], xpad_hbm.at[dst_v.at[c]],
                              ssem[slot])

    wsc = pltpu.async_copy(w_v, wpad_hbm.at[wdst_v], ws)

    g0 = gather(0, 0)
    g1 = gather(1, 1)
    g2 = gather(2, 2)
    g0.wait()
    s0 = scatter(0, 0)
    g1.wait()
    s1 = scatter(1, 1)
    s0.wait()
    g3 = gather(3, 0)
    g2.wait()
    s2 = scatter(2, 2)
    g3.wait()
    s3 = scatter(3, 0)
    wsc.wait()
    s1.wait()
    s2.wait()
    s3.wait()

  return k(x2d, tok_g, dst_g, dst_g.reshape(NW, G_NCH * G_CH), w8)


def _grouped_ffn(x_pad, W1, b1, W2, b2, w_pad, eb, active):
  def body(be_ref, act_ref, x_ref, w1_ref, b1_ref, w2_ref, b2_ref, wp_ref,
           y_ref, acc_ref, xv_ref, xsem):
    f = pl.program_id(0)
    b = pl.program_id(1)

    @pl.when((f == 0) & (b == 0))
    def _():
      pltpu.make_async_copy(x_ref, xv_ref, xsem).start()
      pltpu.make_async_copy(x_ref, xv_ref, xsem).wait()

    @pl.when(act_ref[b] == 1)
    def _():
      sl = pl.ds(b * BLK, BLK)
      xb = xv_ref[sl, :].astype(jnp.float32)
      h = jnp.dot(xb, w1_ref[0],
                  preferred_element_type=jnp.float32) + b1_ref[0, 0]
      h = jnp.maximum(h, 0.0)
      part = jnp.dot(h, w2_ref[0], preferred_element_type=jnp.float32)

      @pl.when(f == 0)
      def _():
        acc_ref[sl, :] = part.astype(jnp.bfloat16)

      @pl.when(f > 0)
      def _():
        acc_ref[sl, :] = (acc_ref[sl, :].astype(jnp.float32)
                          + part).astype(jnp.bfloat16)

      @pl.when(f == NF - 1)
      def _():
        y_ref[...] = ((acc_ref[sl, :].astype(jnp.float32) + b2_ref[0])
                      * wp_ref[:, 0:1])

  grid_spec = pltpu.PrefetchScalarGridSpec(
      num_scalar_prefetch=2,
      grid=(NF, MAXB),
      in_specs=[
          pl.BlockSpec(memory_space=pl.ANY),
          pl.BlockSpec((1, D, F), lambda f, b, be, act: (be[b], 0, f)),
          pl.BlockSpec((1, 1, 1, F), lambda f, b, be, act: (be[b], f, 0, 0)),
          pl.BlockSpec((1, F, D), lambda f, b, be, act: (be[b], f, 0)),
          pl.BlockSpec((1, 1, D), lambda f, b, be, act: (be[b], 0, 0)),
          pl.BlockSpec((BLK, 128), lambda f, b, be, act: (b, 0)),
      ],
      out_specs=pl.BlockSpec(
          (BLK, D),
          lambda f, b, be, act: (jnp.where(f == NF - 1, b, MAXB), 0)),
      scratch_shapes=[pltpu.VMEM((MAXB * BLK, D), jnp.bfloat16),
                      pltpu.VMEM((PAD_N, D), jnp.bfloat16),
                      pltpu.SemaphoreType.DMA],
  )
  return pl.pallas_call(
      body, grid_spec=grid_spec,
      out_shape=jax.ShapeDtypeStruct(((MAXB + 1) * BLK, D), jnp.float32),
  )(eb, active, x_pad, W1, b1.reshape(E, NF, 1, F), W2, b2.reshape(E, 1, D),
    w_pad)


def _combine(y_pad, pp0, pp1):
  mesh = plsc.VectorSubcoreMesh(core_axis_name="c", subcore_axis_name="s")

  @functools.partial(
      pl.kernel, mesh=mesh,
      out_type=jax.ShapeDtypeStruct((S, D), jnp.float32),
      scratch_types=[
          pltpu.VMEM((C_CH,), jnp.int32),
          pltpu.VMEM((C_CH,), jnp.int32),
          pltpu.VMEM((C_CH, D), jnp.float32),
          pltpu.VMEM((C_CH, D), jnp.float32),
          pltpu.SemaphoreType.DMA,
      ],
  )
  def k(y_hbm, pp0_hbm, pp1_hbm, out_hbm, i0_v, i1_v, r0_v, r1_v, sem):
    wid = lax.axis_index("s") * NC + lax.axis_index("c")
    for c in range(C_TOK // C_CH):
      tbase = wid * C_TOK + c * C_CH
      pltpu.sync_copy(pp0_hbm.at[pl.ds(tbase, C_CH)], i0_v)
      pltpu.sync_copy(pp1_hbm.at[pl.ds(tbase, C_CH)], i1_v)
      cp0 = pltpu.async_copy(y_hbm.at[i0_v], r0_v, sem)
      cp1 = pltpu.async_copy(y_hbm.at[i1_v], r1_v, sem)
      cp0.wait()
      cp1.wait()

      def add_body(j, _):
        for kk in range(D // 16):
          r0_v[j, pl.ds(kk * 16, 16)] = (r0_v[j, pl.ds(kk * 16, 16)]
                                         + r1_v[j, pl.ds(kk * 16, 16)])
        return 0

      lax.fori_loop(0, C_CH, add_body, 0)
      pltpu.sync_copy(r0_v, out_hbm.at[pl.ds(tbase, C_CH)])

  return k(y_pad, pp0, pp1)


def kernel(input_emb, Wr, br, W1, b1, W2, b2):
  x2d = input_emb.reshape(S, D)
  e2, w2 = _router(x2d, Wr, br)
  tok_g, dst_g, w8, pp0, pp1, eb, active = _dispatch_tables(e2, w2)
  x_i32 = lax.bitcast_convert_type(
      x2d.astype(jnp.bfloat16).reshape(S, D // 2, 2), jnp.int32)
  xpad_i32, w_pad = _gather_dispatch(x_i32, tok_g, dst_g, w8)
  x_pad = lax.bitcast_convert_type(xpad_i32, jnp.bfloat16).reshape(PAD_N, D)
  y_pad = _grouped_ffn(x_pad, W1, b1, W2, b2, w_pad, eb, active)
  out2d = _combine(y_pad, pp0, pp1)
  return out2d.reshape(1, S, D)


# speedup vs baseline: 1.7765x; 1.7765x over previous
import functools

import jax
import jax.numpy as jnp
from jax import lax
from jax.experimental import pallas as pl
from jax.experimental.pallas import tpu as pltpu
from jax.experimental.pallas import tpu_sc as plsc

S = 2048
D = 1024
E = 8
H = 4096
TOPK = 2
A = S * TOPK

BLK = 256
MAXB = A // BLK + E
PAD_N = MAXB * BLK
F = 2048
NF = H // F

NC = 2
NS = 16
NW = NC * NS

G_CH = 32
G_NCH = (A // NW) // G_CH
C_TOK = S // NW
C_CH = 32


def _router(x2d, Wr, br):
  def body(x_ref, wr_ref, br_ref, e_ref, w_ref):
    logits = jnp.dot(x_ref[...], wr_ref[...],
                     preferred_element_type=jnp.float32) + br_ref[...]
    ids = lax.broadcasted_iota(jnp.int32, (S, E), 1)
    neg = jnp.float32(-3.0e38)
    m0 = jnp.max(logits, axis=-1, keepdims=True)
    i0 = jnp.min(jnp.where(logits == m0, ids, E), axis=-1, keepdims=True)
    masked = jnp.where(ids == i0, neg, logits)
    m1 = jnp.max(masked, axis=-1, keepdims=True)
    i1 = jnp.min(jnp.where(masked == m1, ids, E), axis=-1, keepdims=True)
    w0 = 1.0 / (1.0 + jnp.exp(m1 - m0))
    e_ref[...] = jnp.concatenate([i0, i1], axis=1)
    w_ref[...] = jnp.concatenate([w0, 1.0 - w0], axis=1)

  out_shape = (
      jax.ShapeDtypeStruct((S, TOPK), jnp.int32),
      jax.ShapeDtypeStruct((S, TOPK), jnp.float32),
  )
  return pl.pallas_call(body, out_shape=out_shape)(x2d, Wr, br.reshape(1, E))


def _dispatch_tables(e2, w2):
  i32 = jnp.int32
  flat_e = e2.reshape(A)
  flat_w = w2.reshape(A)
  onehot = (flat_e[:, None] == jnp.arange(E, dtype=i32)[None, :]).astype(i32)
  ccum = jnp.cumsum(onehot, axis=0)
  g = ccum[-1]
  rank = jnp.take_along_axis(ccum, flat_e[:, None], axis=1)[:, 0] - 1
  nblk = (g + BLK - 1) // BLK
  bcum = jnp.cumsum(nblk).astype(i32)
  bcum_ex = jnp.concatenate([jnp.zeros(1, i32), bcum[:-1]])
  total_blocks = bcum[-1]

  b_ids = jnp.arange(MAXB, dtype=i32)
  eb = (b_ids[:, None] >= bcum[None, :]).sum(axis=1, dtype=i32)
  active = (b_ids < total_blocks).astype(i32)
  e_last = jnp.max(jnp.where(g > 0, jnp.arange(E, dtype=i32), 0))
  eb_safe = jnp.where(active == 1, jnp.clip(eb, 0, E - 1), e_last)

  dst_pad = (bcum_ex[flat_e] + rank // BLK) * BLK + rank % BLK

  pp = dst_pad.reshape(S, TOPK)
  tok_g = (jnp.arange(A, dtype=i32) // TOPK).reshape(NW, G_NCH, G_CH)
  dst_g = dst_pad.reshape(NW, G_NCH, G_CH)
  w8 = jnp.broadcast_to(flat_w[:, None], (A, 128))
  return tok_g, dst_g, w8, pp[:, 0], pp[:, 1], eb_safe, active


def _gather_dispatch(x2d, tok_g, dst_g, w8):
  mesh = plsc.VectorSubcoreMesh(core_axis_name="c", subcore_axis_name="s")

  @functools.partial(
      pl.kernel, mesh=mesh,
      out_type=(jax.ShapeDtypeStruct((PAD_N, D), jnp.float32),
                jax.ShapeDtypeStruct((PAD_N, 128), jnp.float32)),
      scratch_types=[
          pltpu.VMEM((G_NCH, G_CH), jnp.int32),
          pltpu.VMEM((G_NCH, G_CH), jnp.int32),
          pltpu.VMEM((G_NCH * G_CH,), jnp.int32),
          pltpu.VMEM((G_NCH * G_CH, 128), jnp.float32),
          pltpu.VMEM((G_CH, D), jnp.float32),
          pltpu.VMEM((G_CH, D), jnp.float32),
          pltpu.VMEM((G_CH, D), jnp.float32),
          pltpu.SemaphoreType.DMA,
          pltpu.SemaphoreType.DMA,
          pltpu.SemaphoreType.DMA,
          pltpu.SemaphoreType.DMA,
          pltpu.SemaphoreType.DMA,
          pltpu.SemaphoreType.DMA,
          pltpu.SemaphoreType.DMA,
      ],
  )
  def k(x_hbm, tok_hbm, dst_hbm, dstf_hbm, w8_hbm, xpad_hbm, wpad_hbm,
        tok_v, dst_v, wdst_v, w_v, r0, r1, r2, gs0, gs1, gs2, ss0, ss1, ss2,
        ws):
    wid = lax.axis_index("s") * NC + lax.axis_index("c")
    pltpu.sync_copy(tok_hbm.at[wid], tok_v)
    pltpu.sync_copy(dst_hbm.at[wid], dst_v)
    pltpu.sync_copy(dstf_hbm.at[wid], wdst_v)
    pltpu.sync_copy(w8_hbm.at[pl.ds(wid * (G_NCH * G_CH), G_NCH * G_CH)], w_v)
    rows = (r0, r1, r2)
    gsem = (gs0, gs1, gs2)
    ssem = (ss0, ss1, ss2)

    def gather(c, slot):
      return pltpu.async_copy(x_hbm.at[tok_v.at[c]], rows[slot], gsem[slot])

    def scatter(c, slot):
      return pltpu.async_copy(rows[slot], xpad_hbm.at[dst_v.at[c]],
                              ssem[slot])

    wsc = pltpu.async_copy(w_v, wpad_hbm.at[wdst_v], ws)

    g0 = gather(0, 0)
    g1 = gather(1, 1)
    g2 = gather(2, 2)
    g0.wait()
    s0 = scatter(0, 0)
    g1.wait()
    s1 = scatter(1, 1)
    s0.wait()
    g3 = gather(3, 0)
    g2.wait()
    s2 = scatter(2, 2)
    g3.wait()
    s3 = scatter(3, 0)
    wsc.wait()
    s1.wait()
    s2.wait()
    s3.wait()

  return k(x2d, tok_g, dst_g, dst_g.reshape(NW, G_NCH * G_CH), w8)


def _grouped_ffn(x_pad, W1, b1, W2, b2, w_pad, eb, active):
  def body(be_ref, act_ref, x_ref, w1_ref, b1_ref, w2_ref, b2_ref, wp_ref,
           y_ref, acc_ref):
    f = pl.program_id(0)
    b = pl.program_id(1)

    @pl.when(act_ref[b] == 1)
    def _():
      sl = pl.ds(b * BLK, BLK)
      h = jnp.dot(x_ref[...], w1_ref[0],
                  preferred_element_type=jnp.float32) + b1_ref[0, 0]
      h = jnp.maximum(h, 0.0)
      part = jnp.dot(h, w2_ref[0], preferred_element_type=jnp.float32)

      @pl.when(f == 0)
      def _():
        acc_ref[sl, :] = part.astype(jnp.bfloat16)

      @pl.when(f > 0)
      def _():
        acc_ref[sl, :] = (acc_ref[sl, :].astype(jnp.float32)
                          + part).astype(jnp.bfloat16)

      @pl.when(f == NF - 1)
      def _():
        y_ref[...] = ((acc_ref[sl, :].astype(jnp.float32) + b2_ref[0])
                      * wp_ref[:, 0:1])

  grid_spec = pltpu.PrefetchScalarGridSpec(
      num_scalar_prefetch=2,
      grid=(NF, MAXB),
      in_specs=[
          pl.BlockSpec((BLK, D), lambda f, b, be, act: (b, 0)),
          pl.BlockSpec((1, D, F), lambda f, b, be, act: (be[b], 0, f)),
          pl.BlockSpec((1, 1, 1, F), lambda f, b, be, act: (be[b], f, 0, 0)),
          pl.BlockSpec((1, F, D), lambda f, b, be, act: (be[b], f, 0)),
          pl.BlockSpec((1, 1, D), lambda f, b, be, act: (be[b], 0, 0)),
          pl.BlockSpec((BLK, 128), lambda f, b, be, act: (b, 0)),
      ],
      out_specs=pl.BlockSpec(
          (BLK, D),
          lambda f, b, be, act: (jnp.where(f == NF - 1, b, MAXB), 0)),
      scratch_shapes=[pltpu.VMEM((MAXB * BLK, D), jnp.bfloat16)],
  )
  return pl.pallas_call(
      body, grid_spec=grid_spec,
      out_shape=jax.ShapeDtypeStruct(((MAXB + 1) * BLK, D), jnp.float32),
  )(eb, active, x_pad, W1, b1.reshape(E, NF, 1, F), W2, b2.reshape(E, 1, D),
    w_pad)


def _combine(y_pad, pp0, pp1):
  mesh = plsc.VectorSubcoreMesh(core_axis_name="c", subcore_axis_name="s")

  @functools.partial(
      pl.kernel, mesh=mesh,
      out_type=jax.ShapeDtypeStruct((S, D), jnp.float32),
      scratch_types=[
          pltpu.VMEM((C_CH,), jnp.int32),
          pltpu.VMEM((C_CH,), jnp.int32),
          pltpu.VMEM((C_CH, D), jnp.float32),
          pltpu.VMEM((C_CH, D), jnp.float32),
          pltpu.SemaphoreType.DMA,
      ],
  )
  def k(y_hbm, pp0_hbm, pp1_hbm, out_hbm, i0_v, i1_v, r0_v, r1_v, sem):
    wid = lax.axis_index("s") * NC + lax.axis_index("c")
    for c in range(C_TOK // C_CH):
      tbase = wid * C_TOK + c * C_CH
      pltpu.sync_copy(pp0_hbm.at[pl.ds(tbase, C_CH)], i0_v)
      pltpu.sync_copy(pp1_hbm.at[pl.ds(tbase, C_CH)], i1_v)
      cp0 = pltpu.async_copy(y_hbm.at[i0_v], r0_v, sem)
      cp1 = pltpu.async_copy(y_hbm.at[i1_v], r1_v, sem)
      cp0.wait()
      cp1.wait()

      def add_body(j, _):
        for kk in range(D // 16):
          r0_v[j, pl.ds(kk * 16, 16)] = (r0_v[j, pl.ds(kk * 16, 16)]
                                         + r1_v[j, pl.ds(kk * 16, 16)])
        return 0

      lax.fori_loop(0, C_CH, add_body, 0)
      pltpu.sync_copy(r0_v, out_hbm.at[pl.ds(tbase, C_CH)])

  return k(y_pad, pp0, pp1)


def kernel(input_emb, Wr, br, W1, b1, W2, b2):
  x2d = input_emb.reshape(S, D)
  e2, w2 = _router(x2d, Wr, br)
  tok_g, dst_g, w8, pp0, pp1, eb, active = _dispatch_tables(e2, w2)
  x_pad, w_pad = _gather_dispatch(x2d, tok_g, dst_g, w8)
  y_pad = _grouped_ffn(x_pad, W1, b1, W2, b2, w_pad, eb, active)
  out2d = _combine(y_pad, pp0, pp1)
  return out2d.reshape(1, S, D)


# speedup vs baseline: 1.8269x; 1.0283x over previous
import functools

import jax
import jax.numpy as jnp
from jax import lax
from jax.experimental import pallas as pl
from jax.experimental.pallas import tpu as pltpu
from jax.experimental.pallas import tpu_sc as plsc

S = 2048
D = 1024
E = 8
H = 4096
TOPK = 2
A = S * TOPK

BLK = 256
MAXB = A // BLK + E
PAD_N = MAXB * BLK
F = 2048
NF = H // F

NC = 2
NS = 16
NW = NC * NS

G_CH = 32
G_NCH = (A // NW) // G_CH
C_TOK = S // NW
C_CH = 32


def _router(x2d, Wr, br):
  def body(x_ref, wr_ref, br_ref, e_ref, w_ref):
    logits = jnp.dot(x_ref[...], wr_ref[...],
                     preferred_element_type=jnp.float32) + br_ref[...]
    ids = lax.broadcasted_iota(jnp.int32, (S, E), 1)
    neg = jnp.float32(-3.0e38)
    m0 = jnp.max(logits, axis=-1, keepdims=True)
    i0 = jnp.min(jnp.where(logits == m0, ids, E), axis=-1, keepdims=True)
    masked = jnp.where(ids == i0, neg, logits)
    m1 = jnp.max(masked, axis=-1, keepdims=True)
    i1 = jnp.min(jnp.where(masked == m1, ids, E), axis=-1, keepdims=True)
    w0 = 1.0 / (1.0 + jnp.exp(m1 - m0))
    e_ref[...] = jnp.concatenate([i0, i1], axis=1)
    w_ref[...] = jnp.concatenate([w0, 1.0 - w0], axis=1)

  out_shape = (
      jax.ShapeDtypeStruct((S, TOPK), jnp.int32),
      jax.ShapeDtypeStruct((S, TOPK), jnp.float32),
  )
  return pl.pallas_call(body, out_shape=out_shape)(x2d, Wr, br.reshape(1, E))


def _dispatch_tables(e2, w2):
  i32 = jnp.int32
  flat_e = e2.reshape(A)
  flat_w = w2.reshape(A)
  onehot = (flat_e[:, None] == jnp.arange(E, dtype=i32)[None, :]).astype(i32)
  ccum = jnp.cumsum(onehot, axis=0)
  g = ccum[-1]
  rank = jnp.sum(ccum * onehot, axis=1, dtype=i32) - 1
  nblk = (g + BLK - 1) // BLK
  bcum = jnp.cumsum(nblk).astype(i32)
  bcum_ex = jnp.concatenate([jnp.zeros(1, i32), bcum[:-1]])
  total_blocks = bcum[-1]

  b_ids = jnp.arange(MAXB, dtype=i32)
  eb = (b_ids[:, None] >= bcum[None, :]).sum(axis=1, dtype=i32)
  active = (b_ids < total_blocks).astype(i32)
  e_last = jnp.max(jnp.where(g > 0, jnp.arange(E, dtype=i32), 0))
  eb_safe = jnp.where(active == 1, jnp.clip(eb, 0, E - 1), e_last)

  bce = jnp.sum(onehot * bcum_ex[None, :], axis=1, dtype=i32)
  dst_pad = (bce + rank // BLK) * BLK + rank % BLK

  pp = dst_pad.reshape(S, TOPK)
  tok_g = (jnp.arange(A, dtype=i32) // TOPK).reshape(NW, G_NCH, G_CH)
  dst_g = dst_pad.reshape(NW, G_NCH, G_CH)
  w8 = jnp.broadcast_to(flat_w[:, None], (A, 128))
  return tok_g, dst_g, w8, pp[:, 0], pp[:, 1], eb_safe, active


def _gather_dispatch(x2d, tok_g, dst_g, w8):
  mesh = plsc.VectorSubcoreMesh(core_axis_name="c", subcore_axis_name="s")

  @functools.partial(
      pl.kernel, mesh=mesh,
      out_type=(jax.ShapeDtypeStruct((PAD_N, D), jnp.float32),
                jax.ShapeDtypeStruct((PAD_N, 128), jnp.float32)),
      scratch_types=[
          pltpu.VMEM((G_NCH, G_CH), jnp.int32),
          pltpu.VMEM((G_NCH, G_CH), jnp.int32),
          pltpu.VMEM((G_NCH * G_CH,), jnp.int32),
          pltpu.VMEM((G_NCH * G_CH, 128), jnp.float32),
          pltpu.VMEM((G_CH, D), jnp.float32),
          pltpu.VMEM((G_CH, D), jnp.float32),
          pltpu.VMEM((G_CH, D), jnp.float32),
          pltpu.SemaphoreType.DMA,
          pltpu.SemaphoreType.DMA,
          pltpu.SemaphoreType.DMA,
          pltpu.SemaphoreType.DMA,
          pltpu.SemaphoreType.DMA,
          pltpu.SemaphoreType.DMA,
          pltpu.SemaphoreType.DMA,
      ],
  )
  def k(x_hbm, tok_hbm, dst_hbm, dstf_hbm, w8_hbm, xpad_hbm, wpad_hbm,
        tok_v, dst_v, wdst_v, w_v, r0, r1, r2, gs0, gs1, gs2, ss0, ss1, ss2,
        ws):
    wid = lax.axis_index("s") * NC + lax.axis_index("c")
    pltpu.sync_copy(tok_hbm.at[wid], tok_v)
    pltpu.sync_copy(dst_hbm.at[wid], dst_v)
    pltpu.sync_copy(dstf_hbm.at[wid], wdst_v)
    pltpu.sync_copy(w8_hbm.at[pl.ds(wid * (G_NCH * G_CH), G_NCH * G_CH)], w_v)
    rows = (r0, r1, r2)
    gsem = (gs0, gs1, gs2)
    ssem = (ss0, ss1, ss2)

    def gather(c, slot):
      return pltpu.async_copy(x_hbm.at[tok_v.at[c]], rows[slot], gsem[slot])

    def scatter(c, slot):
      return pltpu.async_copy(rows[slot], xpad_hbm.at[dst_v.at[c]],
                              ssem[slot])

    wsc = pltpu.async_copy(w_v, wpad_hbm.at[wdst_v], ws)

    g0 = gather(0, 0)
    g1 = gather(1, 1)
    g2 = gather(2, 2)
    g0.wait()
    s0 = scatter(0, 0)
    g1.wait()
    s1 = scatter(1, 1)
    s0.wait()
    g3 = gather(3, 0)
    g2.wait()
    s2 = scatter(2, 2)
    g3.wait()
    s3 = scatter(3, 0)
    wsc.wait()
    s1.wait()
    s2.wait()
    s3.wait()

  return k(x2d, tok_g, dst_g, dst_g.reshape(NW, G_NCH * G_CH), w8)


def _grouped_ffn(x_pad, W1, b1, W2, b2, w_pad, eb, active):
  def body(be_ref, act_ref, x_ref, w1_ref, b1_ref, w2_ref, b2_ref, wp_ref,
           y_ref, acc_ref):
    f = pl.program_id(0)
    b = pl.program_id(1)

    @pl.when(act_ref[b] == 1)
    def _():
      sl = pl.ds(b * BLK, BLK)
      h = jnp.dot(x_ref[...], w1_ref[0],
                  preferred_element_type=jnp.float32) + b1_ref[0, 0]
      h = jnp.maximum(h, 0.0)
      part = jnp.dot(h, w2_ref[0], preferred_element_type=jnp.float32)

      @pl.when(f == 0)
      def _():
        acc_ref[sl, :] = part.astype(jnp.bfloat16)

      @pl.when(f > 0)
      def _():
        acc_ref[sl, :] = (acc_ref[sl, :].astype(jnp.float32)
                          + part).astype(jnp.bfloat16)

      @pl.when(f == NF - 1)
      def _():
        y_ref[...] = ((acc_ref[sl, :].astype(jnp.float32) + b2_ref[0])
                      * wp_ref[:, 0:1])

  grid_spec = pltpu.PrefetchScalarGridSpec(
      num_scalar_prefetch=2,
      grid=(NF, MAXB),
      in_specs=[
          pl.BlockSpec((BLK, D), lambda f, b, be, act: (b, 0)),
          pl.BlockSpec((1, D, F), lambda f, b, be, act: (be[b], 0, f)),
          pl.BlockSpec((1, 1, 1, F), lambda f, b, be, act: (be[b], f, 0, 0)),
          pl.BlockSpec((1, F, D), lambda f, b, be, act: (be[b], f, 0)),
          pl.BlockSpec((1, 1, D), lambda f, b, be, act: (be[b], 0, 0)),
          pl.BlockSpec((BLK, 128), lambda f, b, be, act: (b, 0)),
      ],
      out_specs=pl.BlockSpec(
          (BLK, D),
          lambda f, b, be, act: (jnp.where(f == NF - 1, b, MAXB), 0)),
      scratch_shapes=[pltpu.VMEM((MAXB * BLK, D), jnp.bfloat16)],
  )
  return pl.pallas_call(
      body, grid_spec=grid_spec,
      out_shape=jax.ShapeDtypeStruct(((MAXB + 1) * BLK, D), jnp.float32),
  )(eb, active, x_pad, W1, b1.reshape(E, NF, 1, F), W2, b2.reshape(E, 1, D),
    w_pad)


def _combine(y_pad, pp0, pp1):
  mesh = plsc.VectorSubcoreMesh(core_axis_name="c", subcore_axis_name="s")

  @functools.partial(
      pl.kernel, mesh=mesh,
      out_type=jax.ShapeDtypeStruct((S, D), jnp.float32),
      scratch_types=[
          pltpu.VMEM((C_CH,), jnp.int32),
          pltpu.VMEM((C_CH,), jnp.int32),
          pltpu.VMEM((C_CH, D), jnp.float32),
          pltpu.VMEM((C_CH, D), jnp.float32),
          pltpu.SemaphoreType.DMA,
      ],
  )
  def k(y_hbm, pp0_hbm, pp1_hbm, out_hbm, i0_v, i1_v, r0_v, r1_v, sem):
    wid = lax.axis_index("s") * NC + lax.axis_index("c")
    for c in range(C_TOK // C_CH):
      tbase = wid * C_TOK + c * C_CH
      pltpu.sync_copy(pp0_hbm.at[pl.ds(tbase, C_CH)], i0_v)
      pltpu.sync_copy(pp1_hbm.at[pl.ds(tbase, C_CH)], i1_v)
      cp0 = pltpu.async_copy(y_hbm.at[i0_v], r0_v, sem)
      cp1 = pltpu.async_copy(y_hbm.at[i1_v], r1_v, sem)
      cp0.wait()
      cp1.wait()

      def add_body(j, _):
        for kk in range(D // 16):
          r0_v[j, pl.ds(kk * 16, 16)] = (r0_v[j, pl.ds(kk * 16, 16)]
                                         + r1_v[j, pl.ds(kk * 16, 16)])
        return 0

      lax.fori_loop(0, C_CH, add_body, 0)
      pltpu.sync_copy(r0_v, out_hbm.at[pl.ds(tbase, C_CH)])

  return k(y_pad, pp0, pp1)


def kernel(input_emb, Wr, br, W1, b1, W2, b2):
  x2d = input_emb.reshape(S, D)
  e2, w2 = _router(x2d, Wr, br)
  tok_g, dst_g, w8, pp0, pp1, eb, active = _dispatch_tables(e2, w2)
  x_pad, w_pad = _gather_dispatch(x2d, tok_g, dst_g, w8)
  y_pad = _grouped_ffn(x_pad, W1, b1, W2, b2, w_pad, eb, active)
  out2d = _combine(y_pad, pp0, pp1)
  return out2d.reshape(1, S, D)


# speedup vs baseline: 1.8305x; 1.0020x over previous
import functools

import jax
import jax.numpy as jnp
from jax import lax
from jax.experimental import pallas as pl
from jax.experimental.pallas import tpu as pltpu
from jax.experimental.pallas import tpu_sc as plsc

S = 2048
D = 1024
E = 8
H = 4096
TOPK = 2
A = S * TOPK

BLK = 256
MAXB = A // BLK + E
PAD_N = MAXB * BLK
F = 2048
NF = H // F

NC = 2
NS = 16
NW = NC * NS

G_CH = 32
G_NCH = (A // NW) // G_CH
C_TOK = S // NW
C_CH = 32


def _router(x2d, Wr, br):
  def body(x_ref, wr_ref, br_ref, e_ref, w_ref):
    logits = jnp.dot(x_ref[...], wr_ref[...],
                     preferred_element_type=jnp.float32) + br_ref[...]
    ids = lax.broadcasted_iota(jnp.int32, (S, E), 1)
    neg = jnp.float32(-3.0e38)
    m0 = jnp.max(logits, axis=-1, keepdims=True)
    i0 = jnp.min(jnp.where(logits == m0, ids, E), axis=-1, keepdims=True)
    masked = jnp.where(ids == i0, neg, logits)
    m1 = jnp.max(masked, axis=-1, keepdims=True)
    i1 = jnp.min(jnp.where(masked == m1, ids, E), axis=-1, keepdims=True)
    w0 = 1.0 / (1.0 + jnp.exp(m1 - m0))
    e_ref[...] = jnp.concatenate([i0, i1], axis=1)
    w_ref[...] = jnp.concatenate([w0, 1.0 - w0], axis=1)

  out_shape = (
      jax.ShapeDtypeStruct((S, TOPK), jnp.int32),
      jax.ShapeDtypeStruct((S, TOPK), jnp.float32),
  )
  return pl.pallas_call(body, out_shape=out_shape)(x2d, Wr, br.reshape(1, E))


def _dispatch_tables(e2, w2):
  i32 = jnp.int32
  flat_e = e2.reshape(A)
  flat_w = w2.reshape(A)
  onehot = (flat_e[:, None] == jnp.arange(E, dtype=i32)[None, :]).astype(i32)
  ccum = jnp.cumsum(onehot, axis=0)
  g = ccum[-1]
  rank = jnp.sum(ccum * onehot, axis=1, dtype=i32) - 1
  nblk = (g + BLK - 1) // BLK
  bcum = jnp.cumsum(nblk).astype(i32)
  bcum_ex = jnp.concatenate([jnp.zeros(1, i32), bcum[:-1]])
  total_blocks = bcum[-1]

  b_ids = jnp.arange(MAXB, dtype=i32)
  eb = (b_ids[:, None] >= bcum[None, :]).sum(axis=1, dtype=i32)
  active = (b_ids < total_blocks).astype(i32)
  e_last = jnp.max(jnp.where(g > 0, jnp.arange(E, dtype=i32), 0))
  eb_safe = jnp.where(active == 1, jnp.clip(eb, 0, E - 1), e_last)

  bce = jnp.sum(onehot * bcum_ex[None, :], axis=1, dtype=i32)
  dst_pad = (bce + rank // BLK) * BLK + rank % BLK

  pp = dst_pad.reshape(S, TOPK)
  tok_g = (jnp.arange(A, dtype=i32) // TOPK).reshape(NW, G_NCH, G_CH)
  dst_g = dst_pad.reshape(NW, G_NCH, G_CH)
  w8 = jnp.broadcast_to(flat_w[:, None], (A, 128))
  return tok_g, dst_g, w8, pp[:, 0], pp[:, 1], eb_safe, active


def _gather_dispatch(x2d, tok_g, dst_g, w8):
  mesh = plsc.VectorSubcoreMesh(core_axis_name="c", subcore_axis_name="s")

  @functools.partial(
      pl.kernel, mesh=mesh,
      out_type=(jax.ShapeDtypeStruct((PAD_N, D), jnp.float32),
                jax.ShapeDtypeStruct((PAD_N, 128), jnp.float32)),
      scratch_types=[
          pltpu.VMEM((G_NCH, G_CH), jnp.int32),
          pltpu.VMEM((G_NCH, G_CH), jnp.int32),
          pltpu.VMEM((G_NCH * G_CH,), jnp.int32),
          pltpu.VMEM((G_NCH * G_CH, 128), jnp.float32),
          pltpu.VMEM((G_CH, D), jnp.float32),
          pltpu.VMEM((G_CH, D), jnp.float32),
          pltpu.VMEM((G_CH, D), jnp.float32),
          pltpu.SemaphoreType.DMA,
          pltpu.SemaphoreType.DMA,
          pltpu.SemaphoreType.DMA,
          pltpu.SemaphoreType.DMA,
          pltpu.SemaphoreType.DMA,
          pltpu.SemaphoreType.DMA,
          pltpu.SemaphoreType.DMA,
      ],
  )
  def k(x_hbm, tok_hbm, dst_hbm, dstf_hbm, w8_hbm, xpad_hbm, wpad_hbm,
        tok_v, dst_v, wdst_v, w_v, r0, r1, r2, gs0, gs1, gs2, ss0, ss1, ss2,
        ws):
    wid = lax.axis_index("s") * NC + lax.axis_index("c")
    pltpu.sync_copy(tok_hbm.at[wid], tok_v)
    pltpu.sync_copy(dst_hbm.at[wid], dst_v)
    pltpu.sync_copy(dstf_hbm.at[wid], wdst_v)
    pltpu.sync_copy(w8_hbm.at[pl.ds(wid * (G_NCH * G_CH), G_NCH * G_CH)], w_v)
    rows = (r0, r1, r2)
    gsem = (gs0, gs1, gs2)
    ssem = (ss0, ss1, ss2)

    def gather(c, slot):
      return pltpu.async_copy(x_hbm.at[tok_v.at[c]], rows[slot], gsem[slot])

    def scatter(c, slot):
      return pltpu.async_copy(rows[slot], xpad_hbm.at[dst_v.at[c]],
                              ssem[slot])

    wsc = pltpu.async_copy(w_v, wpad_hbm.at[wdst_v], ws)

    g0 = gather(0, 0)
    g1 = gather(1, 1)
    g2 = gather(2, 2)
    g0.wait()
    s0 = scatter(0, 0)
    g1.wait()
    s1 = scatter(1, 1)
    s0.wait()
    g3 = gather(3, 0)
    g2.wait()
    s2 = scatter(2, 2)
    g3.wait()
    s3 = scatter(3, 0)
    wsc.wait()
    s1.wait()
    s2.wait()
    s3.wait()

  return k(x2d, tok_g, dst_g, dst_g.reshape(NW, G_NCH * G_CH), w8)


def _grouped_ffn(x_pad, W1, b1, W2, b2, w_pad, eb, active):
  def body(be_ref, act_ref, x_ref, w1_ref, b1_ref, w2_ref, b2_ref, wp_ref,
           y_ref, *maybe_acc):
    acc_ref = maybe_acc[0] if maybe_acc else None
    f = pl.program_id(0)
    b = pl.program_id(1)

    @pl.when(act_ref[b] == 1)
    def _():
      h = jnp.dot(x_ref[...], w1_ref[0],
                  preferred_element_type=jnp.float32) + b1_ref[0, 0]
      h = jnp.maximum(h, 0.0)
      part = jnp.dot(h, w2_ref[0], preferred_element_type=jnp.float32)

      if NF == 1:
        y_ref[...] = (part + b2_ref[0]) * wp_ref[:, 0:1]
      else:
        sl = pl.ds(b * BLK, BLK)

        @pl.when(f == 0)
        def _():
          acc_ref[sl, :] = part.astype(jnp.bfloat16)

        @pl.when(f > 0)
        def _():
          acc_ref[sl, :] = (acc_ref[sl, :].astype(jnp.float32)
                            + part).astype(jnp.bfloat16)

        @pl.when(f == NF - 1)
        def _():
          y_ref[...] = ((acc_ref[sl, :].astype(jnp.float32) + b2_ref[0])
                        * wp_ref[:, 0:1])

  grid_spec = pltpu.PrefetchScalarGridSpec(
      num_scalar_prefetch=2,
      grid=(NF, MAXB),
      in_specs=[
          pl.BlockSpec((BLK, D), lambda f, b, be, act: (b, 0)),
          pl.BlockSpec((1, D, F), lambda f, b, be, act: (be[b], 0, f)),
          pl.BlockSpec((1, 1, 1, F), lambda f, b, be, act: (be[b], f, 0, 0)),
          pl.BlockSpec((1, F, D), lambda f, b, be, act: (be[b], f, 0)),
          pl.BlockSpec((1, 1, D), lambda f, b, be, act: (be[b], 0, 0)),
          pl.BlockSpec((BLK, 128), lambda f, b, be, act: (b, 0)),
      ],
      out_specs=pl.BlockSpec(
          (BLK, D),
          lambda f, b, be, act: (jnp.where(f == NF - 1, b, MAXB), 0)),
      scratch_shapes=([pltpu.VMEM((MAXB * BLK, D), jnp.bfloat16)]
                      if NF > 1 else []),
  )
  return pl.pallas_call(
      body, grid_spec=grid_spec,
      out_shape=jax.ShapeDtypeStruct(((MAXB + 1) * BLK, D), jnp.float32),
  )(eb, active, x_pad, W1, b1.reshape(E, NF, 1, F), W2, b2.reshape(E, 1, D),
    w_pad)


def _combine(y_pad, pp0, pp1):
  mesh = plsc.VectorSubcoreMesh(core_axis_name="c", subcore_axis_name="s")

  @functools.partial(
      pl.kernel, mesh=mesh,
      out_type=jax.ShapeDtypeStruct((S, D), jnp.float32),
      scratch_types=[
          pltpu.VMEM((C_CH,), jnp.int32),
          pltpu.VMEM((C_CH,), jnp.int32),
          pltpu.VMEM((C_CH, D), jnp.float32),
          pltpu.VMEM((C_CH, D), jnp.float32),
          pltpu.SemaphoreType.DMA,
      ],
  )
  def k(y_hbm, pp0_hbm, pp1_hbm, out_hbm, i0_v, i1_v, r0_v, r1_v, sem):
    wid = lax.axis_index("s") * NC + lax.axis_index("c")
    for c in range(C_TOK // C_CH):
      tbase = wid * C_TOK + c * C_CH
      pltpu.sync_copy(pp0_hbm.at[pl.ds(tbase, C_CH)], i0_v)
      pltpu.sync_copy(pp1_hbm.at[pl.ds(tbase, C_CH)], i1_v)
      cp0 = pltpu.async_copy(y_hbm.at[i0_v], r0_v, sem)
      cp1 = pltpu.async_copy(y_hbm.at[i1_v], r1_v, sem)
      cp0.wait()
      cp1.wait()

      def add_body(j, _):
        for kk in range(D // 16):
          r0_v[j, pl.ds(kk * 16, 16)] = (r0_v[j, pl.ds(kk * 16, 16)]
                                         + r1_v[j, pl.ds(kk * 16, 16)])
        return 0

      lax.fori_loop(0, C_CH, add_body, 0)
      pltpu.sync_copy(r0_v, out_hbm.at[pl.ds(tbase, C_CH)])

  return k(y_pad, pp0, pp1)


def kernel(input_emb, Wr, br, W1, b1, W2, b2):
  x2d = input_emb.reshape(S, D)
  e2, w2 = _router(x2d, Wr, br)
  tok_g, dst_g, w8, pp0, pp1, eb, active = _dispatch_tables(e2, w2)
  x_pad, w_pad = _gather_dispatch(x2d, tok_g, dst_g, w8)
  y_pad = _grouped_ffn(x_pad, W1, b1, W2, b2, w_pad, eb, active)
  out2d = _combine(y_pad, pp0, pp1)
  return out2d.reshape(1, S, D)


# speedup vs baseline: 1.9736x; 1.0782x over previous
import functools

import jax
import jax.numpy as jnp
from jax import lax
from jax.experimental import pallas as pl
from jax.experimental.pallas import tpu as pltpu
from jax.experimental.pallas import tpu_sc as plsc

S = 2048
D = 1024
E = 8
H = 4096
TOPK = 2
A = S * TOPK

BLK = 512
MAXB = A // BLK + E
PAD_N = MAXB * BLK
F = 2048
NF = H // F

NC = 2
NS = 16
NW = NC * NS

G_CH = 32
G_NCH = (A // NW) // G_CH
C_TOK = S // NW
C_CH = 32


def _router(x2d, Wr, br):
  def body(x_ref, wr_ref, br_ref, e_ref, w_ref):
    logits = jnp.dot(x_ref[...], wr_ref[...],
                     preferred_element_type=jnp.float32) + br_ref[...]
    ids = lax.broadcasted_iota(jnp.int32, (S, E), 1)
    neg = jnp.float32(-3.0e38)
    m0 = jnp.max(logits, axis=-1, keepdims=True)
    i0 = jnp.min(jnp.where(logits == m0, ids, E), axis=-1, keepdims=True)
    masked = jnp.where(ids == i0, neg, logits)
    m1 = jnp.max(masked, axis=-1, keepdims=True)
    i1 = jnp.min(jnp.where(masked == m1, ids, E), axis=-1, keepdims=True)
    w0 = 1.0 / (1.0 + jnp.exp(m1 - m0))
    e_ref[...] = jnp.concatenate([i0, i1], axis=1)
    w_ref[...] = jnp.concatenate([w0, 1.0 - w0], axis=1)

  out_shape = (
      jax.ShapeDtypeStruct((S, TOPK), jnp.int32),
      jax.ShapeDtypeStruct((S, TOPK), jnp.float32),
  )
  return pl.pallas_call(body, out_shape=out_shape)(x2d, Wr, br.reshape(1, E))


def _dispatch_tables(e2, w2):
  i32 = jnp.int32
  flat_e = e2.reshape(A)
  flat_w = w2.reshape(A)
  onehot = (flat_e[:, None] == jnp.arange(E, dtype=i32)[None, :]).astype(i32)
  ccum = jnp.cumsum(onehot, axis=0)
  g = ccum[-1]
  rank = jnp.sum(ccum * onehot, axis=1, dtype=i32) - 1
  nblk = (g + BLK - 1) // BLK
  bcum = jnp.cumsum(nblk).astype(i32)
  bcum_ex = jnp.concatenate([jnp.zeros(1, i32), bcum[:-1]])
  total_blocks = bcum[-1]

  b_ids = jnp.arange(MAXB, dtype=i32)
  eb = (b_ids[:, None] >= bcum[None, :]).sum(axis=1, dtype=i32)
  active = (b_ids < total_blocks).astype(i32)
  e_last = jnp.max(jnp.where(g > 0, jnp.arange(E, dtype=i32), 0))
  eb_safe = jnp.where(active == 1, jnp.clip(eb, 0, E - 1), e_last)

  bce = jnp.sum(onehot * bcum_ex[None, :], axis=1, dtype=i32)
  dst_pad = (bce + rank // BLK) * BLK + rank % BLK

  pp = dst_pad.reshape(S, TOPK)
  tok_g = (jnp.arange(A, dtype=i32) // TOPK).reshape(NW, G_NCH, G_CH)
  dst_g = dst_pad.reshape(NW, G_NCH, G_CH)
  w8 = jnp.broadcast_to(flat_w[:, None], (A, 128))
  return tok_g, dst_g, w8, pp[:, 0], pp[:, 1], eb_safe, active


def _gather_dispatch(x2d, tok_g, dst_g, w8):
  mesh = plsc.VectorSubcoreMesh(core_axis_name="c", subcore_axis_name="s")

  @functools.partial(
      pl.kernel, mesh=mesh,
      out_type=(jax.ShapeDtypeStruct((PAD_N, D), jnp.float32),
                jax.ShapeDtypeStruct((PAD_N, 128), jnp.float32)),
      scratch_types=[
          pltpu.VMEM((G_NCH, G_CH), jnp.int32),
          pltpu.VMEM((G_NCH, G_CH), jnp.int32),
          pltpu.VMEM((G_NCH * G_CH,), jnp.int32),
          pltpu.VMEM((G_NCH * G_CH, 128), jnp.float32),
          pltpu.VMEM((G_CH, D), jnp.float32),
          pltpu.VMEM((G_CH, D), jnp.float32),
          pltpu.VMEM((G_CH, D), jnp.float32),
          pltpu.SemaphoreType.DMA,
          pltpu.SemaphoreType.DMA,
          pltpu.SemaphoreType.DMA,
          pltpu.SemaphoreType.DMA,
          pltpu.SemaphoreType.DMA,
          pltpu.SemaphoreType.DMA,
          pltpu.SemaphoreType.DMA,
      ],
  )
  def k(x_hbm, tok_hbm, dst_hbm, dstf_hbm, w8_hbm, xpad_hbm, wpad_hbm,
        tok_v, dst_v, wdst_v, w_v, r0, r1, r2, gs0, gs1, gs2, ss0, ss1, ss2,
        ws):
    wid = lax.axis_index("s") * NC + lax.axis_index("c")
    pltpu.sync_copy(tok_hbm.at[wid], tok_v)
    pltpu.sync_copy(dst_hbm.at[wid], dst_v)
    pltpu.sync_copy(dstf_hbm.at[wid], wdst_v)
    pltpu.sync_copy(w8_hbm.at[pl.ds(wid * (G_NCH * G_CH), G_NCH * G_CH)], w_v)
    rows = (r0, r1, r2)
    gsem = (gs0, gs1, gs2)
    ssem = (ss0, ss1, ss2)

    def gather(c, slot):
      return pltpu.async_copy(x_hbm.at[tok_v.at[c]], rows[slot], gsem[slot])

    def scatter(c, slot):
      return pltpu.async_copy(rows[slot], xpad_hbm.at[dst_v.at[c]],
                              ssem[slot])

    wsc = pltpu.async_copy(w_v, wpad_hbm.at[wdst_v], ws)

    g0 = gather(0, 0)
    g1 = gather(1, 1)
    g2 = gather(2, 2)
    g0.wait()
    s0 = scatter(0, 0)
    g1.wait()
    s1 = scatter(1, 1)
    s0.wait()
    g3 = gather(3, 0)
    g2.wait()
    s2 = scatter(2, 2)
    g3.wait()
    s3 = scatter(3, 0)
    wsc.wait()
    s1.wait()
    s2.wait()
    s3.wait()

  return k(x2d, tok_g, dst_g, dst_g.reshape(NW, G_NCH * G_CH), w8)


def _grouped_ffn(x_pad, W1, b1, W2, b2, w_pad, eb, active):
  def body(be_ref, act_ref, x_ref, w1_ref, b1_ref, w2_ref, b2_ref, wp_ref,
           y_ref, *maybe_acc):
    acc_ref = maybe_acc[0] if maybe_acc else None
    f = pl.program_id(0)
    b = pl.program_id(1)

    @pl.when(act_ref[b] == 1)
    def _():
      h = jnp.dot(x_ref[...], w1_ref[0],
                  preferred_element_type=jnp.float32) + b1_ref[0, 0]
      h = jnp.maximum(h, 0.0)
      part = jnp.dot(h, w2_ref[0], preferred_element_type=jnp.float32)

      if NF == 1:
        y_ref[...] = (part + b2_ref[0]) * wp_ref[:, 0:1]
      else:
        sl = pl.ds(b * BLK, BLK)

        @pl.when(f == 0)
        def _():
          acc_ref[sl, :] = part.astype(jnp.bfloat16)

        @pl.when(f > 0)
        def _():
          acc_ref[sl, :] = (acc_ref[sl, :].astype(jnp.float32)
                            + part).astype(jnp.bfloat16)

        @pl.when(f == NF - 1)
        def _():
          y_ref[...] = ((acc_ref[sl, :].astype(jnp.float32) + b2_ref[0])
                        * wp_ref[:, 0:1])

  grid_spec = pltpu.PrefetchScalarGridSpec(
      num_scalar_prefetch=2,
      grid=(NF, MAXB),
      in_specs=[
          pl.BlockSpec((BLK, D), lambda f, b, be, act: (b, 0)),
          pl.BlockSpec((1, D, F), lambda f, b, be, act: (be[b], 0, f)),
          pl.BlockSpec((1, 1, 1, F), lambda f, b, be, act: (be[b], f, 0, 0)),
          pl.BlockSpec((1, F, D), lambda f, b, be, act: (be[b], f, 0)),
          pl.BlockSpec((1, 1, D), lambda f, b, be, act: (be[b], 0, 0)),
          pl.BlockSpec((BLK, 128), lambda f, b, be, act: (b, 0)),
      ],
      out_specs=pl.BlockSpec(
          (BLK, D),
          lambda f, b, be, act: (jnp.where(f == NF - 1, b, MAXB), 0)),
      scratch_shapes=([pltpu.VMEM((MAXB * BLK, D), jnp.bfloat16)]
                      if NF > 1 else []),
  )
  return pl.pallas_call(
      body, grid_spec=grid_spec,
      compiler_params=pltpu.CompilerParams(
          vmem_limit_bytes=100 * 1024 * 1024),
      out_shape=jax.ShapeDtypeStruct(((MAXB + 1) * BLK, D), jnp.float32),
  )(eb, active, x_pad, W1, b1.reshape(E, NF, 1, F), W2, b2.reshape(E, 1, D),
    w_pad)


def _combine(y_pad, pp0, pp1):
  mesh = plsc.VectorSubcoreMesh(core_axis_name="c", subcore_axis_name="s")

  @functools.partial(
      pl.kernel, mesh=mesh,
      out_type=jax.ShapeDtypeStruct((S, D), jnp.float32),
      scratch_types=[
          pltpu.VMEM((C_CH,), jnp.int32),
          pltpu.VMEM((C_CH,), jnp.int32),
          pltpu.VMEM((C_CH, D), jnp.float32),
          pltpu.VMEM((C_CH, D), jnp.float32),
          pltpu.SemaphoreType.DMA,
      ],
  )
  def k(y_hbm, pp0_hbm, pp1_hbm, out_hbm, i0_v, i1_v, r0_v, r1_v, sem):
    wid = lax.axis_index("s") * NC + lax.axis_index("c")
    for c in range(C_TOK // C_CH):
      tbase = wid * C_TOK + c * C_CH
      pltpu.sync_copy(pp0_hbm.at[pl.ds(tbase, C_CH)], i0_v)
      pltpu.sync_copy(pp1_hbm.at[pl.ds(tbase, C_CH)], i1_v)
      cp0 = pltpu.async_copy(y_hbm.at[i0_v], r0_v, sem)
      cp1 = pltpu.async_copy(y_hbm.at[i1_v], r1_v, sem)
      cp0.wait()
      cp1.wait()

      def add_body(j, _):
        for kk in range(D // 16):
          r0_v[j, pl.ds(kk * 16, 16)] = (r0_v[j, pl.ds(kk * 16, 16)]
                                         + r1_v[j, pl.ds(kk * 16, 16)])
        return 0

      lax.fori_loop(0, C_CH, add_body, 0)
      pltpu.sync_copy(r0_v, out_hbm.at[pl.ds(tbase, C_CH)])

  return k(y_pad, pp0, pp1)


def kernel(input_emb, Wr, br, W1, b1, W2, b2):
  x2d = input_emb.reshape(S, D)
  e2, w2 = _router(x2d, Wr, br)
  tok_g, dst_g, w8, pp0, pp1, eb, active = _dispatch_tables(e2, w2)
  x_pad, w_pad = _gather_dispatch(x2d, tok_g, dst_g, w8)
  y_pad = _grouped_ffn(x_pad, W1, b1, W2, b2, w_pad, eb, active)
  out2d = _combine(y_pad, pp0, pp1)
  return out2d.reshape(1, S, D)
